# baseline (speedup 1.0000x reference)
import jax
import jax.numpy as jnp
from jax.experimental import pallas as pl

_EPS = 1e-12


def _aux_loss_body(cls_ref, bp_ref, bt_ref, pk_ref, out_ref):
    j = pl.program_id(1)
    T, C = cls_ref.shape[1], cls_ref.shape[2]

    p = jnp.clip(cls_ref[0], _EPS, 1.0 - _EPS)
    pk = pk_ref[0]
    lab_col = pk[:, 0:1]
    w_col = pk[:, 1:2]
    s_col = pk[:, 2:3]

    cidx = jax.lax.broadcasted_iota(jnp.int32, (T, C), 1).astype(jnp.float32)
    posr = (lab_col >= 0.0) & (lab_col < C)
    mask = (cidx == lab_col) & posr

    logn = jnp.log(1.0 - p)
    logp = jnp.log(p)
    neg = -logn * p * p
    bce = -(s_col * logp + (1.0 - s_col) * logn)
    sf = jnp.abs(s_col - p)
    pos_loss = bce * sf * sf
    loss = jnp.where(mask, pos_loss, neg)
    lc_part = jnp.sum(loss * w_col)

    bp = bp_ref[0]
    bt = bt_ref[0]
    lt = jnp.maximum(bp[:, 0:2], bt[:, 0:2])
    rb = jnp.minimum(bp[:, 2:4], bt[:, 2:4])
    wh = jnp.clip(rb - lt, 0.0, None)
    overlap = wh[:, 0:1] * wh[:, 1:2]
    ap = (bp[:, 2:3] - bp[:, 0:1]) * (bp[:, 3:4] - bp[:, 1:2])
    ag = (bt[:, 2:3] - bt[:, 0:1]) * (bt[:, 3:4] - bt[:, 1:2])
    union = ap + ag - overlap + 1e-7
    ious = overlap / union
    elt = jnp.minimum(bp[:, 0:2], bt[:, 0:2])
    erb = jnp.maximum(bp[:, 2:4], bt[:, 2:4])
    ewh = jnp.clip(erb - elt, 0.0, None)
    enclose = ewh[:, 0:1] * ewh[:, 1:2] + 1e-7
    gious = ious - (enclose - union) / enclose
    gl = 1.0 - gious
    pw = s_col * posr.astype(jnp.float32)
    lb_part = jnp.sum(gl * pw) * 2.0
    caf_part = jnp.sum(s_col)
    baf_part = jnp.sum(pw)

    li = jax.lax.broadcasted_iota(jnp.int32, (1, 1, 4), 2)
    vals = jnp.where(li == 0, lc_part,
                     jnp.where(li == 1, lb_part,
                               jnp.where(li == 2, caf_part, baf_part)))

    @pl.when(j == 0)
    def _():
        out_ref[...] = vals

    @pl.when(j != 0)
    def _():
        out_ref[...] += vals


def _run(cls_scores, bbox_preds, bbox_targets, packed, tile_n, interpret=False):
    B, N, C = cls_scores.shape
    nj = N // tile_n
    return pl.pallas_call(
        _aux_loss_body,
        grid=(B, nj),
        in_specs=[
            pl.BlockSpec((1, tile_n, C), lambda b, j: (b, j, 0)),
            pl.BlockSpec((1, tile_n, 4), lambda b, j: (b, j, 0)),
            pl.BlockSpec((1, tile_n, 4), lambda b, j: (b, j, 0)),
            pl.BlockSpec((1, tile_n, 4), lambda b, j: (b, j, 0)),
        ],
        out_specs=pl.BlockSpec((1, 1, 4), lambda b, j: (b, 0, 0)),
        out_shape=jax.ShapeDtypeStruct((B, 1, 4), jnp.float32),
        interpret=interpret,
    )(cls_scores, bbox_preds, bbox_targets, packed)


def kernel(cls_scores, bbox_preds, labels, label_weights, bbox_targets,
           alignment_metrics, *, tile_n=2000, interpret=False):
    B, N, C = cls_scores.shape
    packed = jnp.stack(
        [labels.astype(jnp.float32), label_weights, alignment_metrics,
         jnp.zeros_like(label_weights)], axis=-1)
    res = _run(cls_scores, bbox_preds, bbox_targets, packed, tile_n,
               interpret=interpret)
    lc = res[:, 0, 0]
    lb = res[:, 0, 1]
    cls_avg = jnp.clip(jnp.sum(res[:, 0, 2]), 1.0, None)
    bbox_avg = jnp.clip(jnp.sum(res[:, 0, 3]), 1.0, None)
    return jnp.stack([lc / cls_avg, lb / bbox_avg])
